# baseline (speedup 1.0000x reference)
import functools

import jax
import jax.numpy as jnp
from jax import lax
from jax.experimental import pallas as pl
from jax.experimental.pallas import tpu as pltpu
from jax.experimental.pallas import tpu_sc as plsc

NC = 2
NS = 16
NW = NC * NS
L = 16
CE = 128
RI = 64


def _mesh():
    return plsc.VectorSubcoreMesh(
        core_axis_name="c", subcore_axis_name="s", num_cores=NC, num_subcores=NS
    )


def _make_deg_kernel(e_pad, n_acc):
    epw = e_pad // NW
    cpw = epw // CE
    rpt = n_acc // NS

    @functools.partial(
        pl.kernel,
        out_type=jax.ShapeDtypeStruct((NC, n_acc), jnp.float32),
        mesh=_mesh(),
        scratch_types=[
            pltpu.VMEM_SHARED((n_acc,), jnp.float32),
            pltpu.VMEM((CE,), jnp.int32),
            pltpu.VMEM((CE,), jnp.float32),
            pltpu.VMEM((rpt,), jnp.float32),
        ],
    )
    def deg_kernel(dst_hbm, out_hbm, acc, id0, ones_v, tmp):
        cid = lax.axis_index("c")
        sid = lax.axis_index("s")
        wid = sid * NC + cid
        base0 = wid * epw

        def fill(i, _):
            tmp[pl.ds(i * L, L)] = jnp.zeros((L,), jnp.float32)
            return 0

        lax.fori_loop(0, rpt // L, fill, 0, unroll=4)

        def fill1(i, _):
            ones_v[pl.ds(i * L, L)] = jnp.ones((L,), jnp.float32)
            return 0

        lax.fori_loop(0, CE // L, fill1, 0, unroll=4)
        pltpu.sync_copy(tmp, acc.at[pl.ds(sid * rpt, rpt)])
        plsc.subcore_barrier()

        def step(c, _):
            pltpu.sync_copy(dst_hbm.at[pl.ds(base0 + c * CE, CE)], id0)
            pltpu.sync_copy(ones_v, acc.at[id0], add=True)
            return 0

        lax.fori_loop(0, cpw, step, 0)
        plsc.subcore_barrier()
        pltpu.sync_copy(acc.at[pl.ds(sid * rpt, rpt)], tmp)
        pltpu.sync_copy(tmp, out_hbm.at[cid, pl.ds(sid * rpt, rpt)])

    return deg_kernel


def _make_scatter_kernel(e_pad, n_acc, d):
    epw = e_pad // NW
    cpw = epw // CE
    rpt = n_acc // NS

    @functools.partial(
        pl.kernel,
        out_type=jax.ShapeDtypeStruct((NC, n_acc, d), jnp.float32),
        mesh=_mesh(),
        scratch_types=[
            pltpu.VMEM_SHARED((n_acc, d), jnp.float32),
            pltpu.VMEM((CE,), jnp.int32),
            pltpu.VMEM((CE,), jnp.int32),
            pltpu.VMEM((CE, d), jnp.float32),
            pltpu.SemaphoreType.DMA,
        ],
    )
    def scatter_kernel(
        h_hbm, src_hbm, dst_hbm, out_hbm, acc, is0, id0, rows0, gsem,
    ):
        cid = lax.axis_index("c")
        sid = lax.axis_index("s")
        wid = sid * NC + cid
        base0 = wid * epw

        def fill(i, _):
            rows0[i // (d // L), pl.ds((i % (d // L)) * L, L)] = jnp.zeros(
                (L,), jnp.float32
            )
            return 0

        lax.fori_loop(0, CE * (d // L), fill, 0, unroll=8)

        def zinit(i, _):
            pltpu.sync_copy(rows0, acc.at[pl.ds(sid * rpt + i * CE, CE), :])
            return 0

        lax.fori_loop(0, rpt // CE, zinit, 0)
        plsc.subcore_barrier()

        def step(c, _):
            base = base0 + c * CE
            pltpu.sync_copy(src_hbm.at[pl.ds(base, CE)], is0)
            pltpu.sync_copy(dst_hbm.at[pl.ds(base, CE)], id0)
            pltpu.async_copy(h_hbm.at[is0], rows0, gsem).wait()
            pltpu.sync_copy(rows0, acc.at[id0], add=True)
            return 0

        lax.fori_loop(0, cpw, step, 0)
        plsc.subcore_barrier()

        def wb(i, _):
            r = sid * rpt + i * CE
            pltpu.sync_copy(acc.at[pl.ds(r, CE), :], rows0)
            pltpu.sync_copy(rows0, out_hbm.at[cid, pl.ds(r, CE), :])
            return 0

        lax.fori_loop(0, rpt // CE, wb, 0)

    return scatter_kernel


def _dinv_of(deg_ref):
    deg = deg_ref[0, :] + deg_ref[1, :] + 1.0
    return lax.rsqrt(deg)[:, None]


def _tc_first(deg_ref, x_ref, w_ref, h_ref):
    dinv = _dinv_of(deg_ref)
    h = jnp.dot(x_ref[...], w_ref[...], preferred_element_type=jnp.float32)
    h_ref[...] = h * dinv


def _tc_mid(n_real, eps, p_ref, hprev_ref, deg_ref, b_ref, g_ref, be_ref, w_ref, h_ref):
    n_pad = hprev_ref.shape[0]
    dinv = _dinv_of(deg_ref)
    agg = p_ref[0] + p_ref[1] + hprev_ref[...]
    y = agg * dinv + b_ref[...][None, :]
    rid = lax.broadcasted_iota(jnp.int32, (n_pad, 1), 0)
    mask = rid < n_real
    y = jnp.where(mask, y, 0.0)
    mu = jnp.sum(y, axis=0, keepdims=True) / n_real
    cent = jnp.where(mask, y - mu, 0.0)
    var = jnp.sum(cent * cent, axis=0, keepdims=True) / n_real
    yn = g_ref[...][None, :] * cent * lax.rsqrt(var + eps) + be_ref[...][None, :]
    z = jnp.where(mask, jnp.maximum(yn, 0.0), 0.0)
    h_ref[...] = jnp.dot(z, w_ref[...], preferred_element_type=jnp.float32) * dinv


def _tc_last(n_real, p_ref, hprev_ref, deg_ref, b_ref, out_ref):
    dinv = _dinv_of(deg_ref)
    agg = p_ref[0] + p_ref[1] + hprev_ref[...]
    y = agg * dinv + b_ref[...][None, :]
    out_ref[...] = y[:n_real, :]


def kernel(x, edge_index, W1, b1, g1, be1, W2, b2, g2, be2, W3, b3):
    n, d = x.shape
    e = edge_index.shape[1]
    n_acc = ((n + 1 + (NS * L) - 1) // (NS * L)) * (NS * L)
    gran = NW * CE * 2
    e_pad = ((e + gran - 1) // gran) * gran

    pad = jnp.full((e_pad - e,), n, dtype=jnp.int32)
    src = jnp.concatenate([edge_index[0], pad])
    dst = jnp.concatenate([edge_index[1], pad])
    x_pad = jnp.pad(x, ((0, n_acc - n), (0, 0)))

    deg_kernel = _make_deg_kernel(e_pad, n_acc)
    scatter_kernel = _make_scatter_kernel(e_pad, n_acc, d)

    deg2 = deg_kernel(dst)

    h1 = pl.pallas_call(
        _tc_first,
        out_shape=jax.ShapeDtypeStruct((n_acc, d), jnp.float32),
    )(deg2, x_pad, W1)

    p1 = scatter_kernel(h1, src, dst)
    h2 = pl.pallas_call(
        functools.partial(_tc_mid, n, 1e-5),
        out_shape=jax.ShapeDtypeStruct((n_acc, d), jnp.float32),
    )(p1, h1, deg2, b1, g1, be1, W2)

    p2 = scatter_kernel(h2, src, dst)
    h3 = pl.pallas_call(
        functools.partial(_tc_mid, n, 1e-5),
        out_shape=jax.ShapeDtypeStruct((n_acc, d), jnp.float32),
    )(p2, h2, deg2, b2, g2, be2, W3)

    p3 = scatter_kernel(h3, src, dst)
    out = pl.pallas_call(
        functools.partial(_tc_last, n),
        out_shape=jax.ShapeDtypeStruct((n, d), jnp.float32),
    )(p3, h3, deg2, b3)
    return out


# speedup vs baseline: 2.3065x; 2.3065x over previous
import functools

import jax
import jax.numpy as jnp
from jax import lax
from jax.experimental import pallas as pl
from jax.experimental.pallas import tpu as pltpu
from jax.experimental.pallas import tpu_sc as plsc

NC = 2
NS = 16
NW = NC * NS
L = 16
CE = 128
RI = 64


def _mesh():
    return plsc.VectorSubcoreMesh(
        core_axis_name="c", subcore_axis_name="s", num_cores=NC, num_subcores=NS
    )


def _make_deg_kernel(e_pad, n_acc):
    epw = e_pad // NW
    cpw = epw // CE
    rpt = n_acc // NS

    @functools.partial(
        pl.kernel,
        out_type=jax.ShapeDtypeStruct((NC, n_acc), jnp.float32),
        mesh=_mesh(),
        scratch_types=[
            pltpu.VMEM_SHARED((n_acc,), jnp.float32),
            pltpu.VMEM((CE,), jnp.int32),
            pltpu.VMEM((CE,), jnp.float32),
            pltpu.VMEM((rpt,), jnp.float32),
        ],
    )
    def deg_kernel(dst_hbm, out_hbm, acc, id0, ones_v, tmp):
        cid = lax.axis_index("c")
        sid = lax.axis_index("s")
        wid = sid * NC + cid
        base0 = wid * epw

        def fill(i, _):
            tmp[pl.ds(i * L, L)] = jnp.zeros((L,), jnp.float32)
            return 0

        lax.fori_loop(0, rpt // L, fill, 0, unroll=4)

        def fill1(i, _):
            ones_v[pl.ds(i * L, L)] = jnp.ones((L,), jnp.float32)
            return 0

        lax.fori_loop(0, CE // L, fill1, 0, unroll=4)
        pltpu.sync_copy(tmp, acc.at[pl.ds(sid * rpt, rpt)])
        plsc.subcore_barrier()

        def step(c, _):
            pltpu.sync_copy(dst_hbm.at[pl.ds(base0 + c * CE, CE)], id0)
            pltpu.sync_copy(ones_v, acc.at[id0], add=True)
            return 0

        lax.fori_loop(0, cpw, step, 0)
        plsc.subcore_barrier()
        pltpu.sync_copy(acc.at[pl.ds(sid * rpt, rpt)], tmp)
        pltpu.sync_copy(tmp, out_hbm.at[cid, pl.ds(sid * rpt, rpt)])

    return deg_kernel


def _make_scatter_kernel(e_pad, n_acc, d):
    epw = e_pad // NW
    cpw = epw // CE
    rpt = n_acc // NS

    @functools.partial(
        pl.kernel,
        out_type=jax.ShapeDtypeStruct((NC, 2, n_acc, d // 2), jnp.float32),
        mesh=_mesh(),
        scratch_types=[
            pltpu.VMEM_SHARED((n_acc, d // 2), jnp.float32),
            pltpu.VMEM_SHARED((n_acc, d // 2), jnp.float32),
            pltpu.VMEM((CE,), jnp.int32),
            pltpu.VMEM((CE,), jnp.int32),
            pltpu.VMEM((CE,), jnp.int32),
            pltpu.VMEM((CE,), jnp.int32),
            pltpu.VMEM((CE, d // 2), jnp.float32),
            pltpu.VMEM((CE, d // 2), jnp.float32),
            pltpu.SemaphoreType.DMA,
            pltpu.SemaphoreType.DMA,
        ],
    )
    def scatter_kernel(
        h_hbm, src_hbm, dst_hbm, out_hbm,
        hsp, acc, is0, is1, id0, id1, rows0, rows1, gsem, ssem,
    ):
        dh = d // 2
        cid = lax.axis_index("c")
        sid = lax.axis_index("s")
        wid = sid * NC + cid
        base0 = wid * epw
        r0 = sid * rpt

        def wait_gather(buf):
            pltpu.make_async_copy(hsp.at[is0], buf, gsem).wait()

        def wait_scatter(buf):
            pltpu.make_async_copy(buf, acc.at[id0], ssem).wait()

        for kk in (0, 1):

            def fill(i, _):
                rows0[i // (dh // L), pl.ds((i % (dh // L)) * L, L)] = jnp.zeros(
                    (L,), jnp.float32
                )
                return 0

            lax.fori_loop(0, CE * (dh // L), fill, 0, unroll=8)

            def zinit(i, _):
                pltpu.sync_copy(rows0, acc.at[pl.ds(r0 + i * CE, CE), :])
                return 0

            lax.fori_loop(0, rpt // CE, zinit, 0)

            def stage(i, _):
                rs = pl.ds(r0 + i * CE, CE)
                pltpu.sync_copy(h_hbm.at[kk, rs, :], rows1)
                pltpu.sync_copy(rows1, hsp.at[rs, :])
                return 0

            lax.fori_loop(0, rpt // CE, stage, 0)
            plsc.subcore_barrier()

            def pair(j, _):
                i0 = j * 2
                i1 = i0 + 1
                pltpu.sync_copy(src_hbm.at[pl.ds(base0 + i0 * CE, CE)], is0)
                pltpu.sync_copy(dst_hbm.at[pl.ds(base0 + i0 * CE, CE)], id0)
                pltpu.async_copy(hsp.at[is0], rows0, gsem)

                @pl.when(j > 0)
                def _():
                    wait_scatter(rows1)

                pltpu.sync_copy(src_hbm.at[pl.ds(base0 + i1 * CE, CE)], is1)
                pltpu.sync_copy(dst_hbm.at[pl.ds(base0 + i1 * CE, CE)], id1)
                wait_gather(rows0)
                pltpu.async_copy(hsp.at[is1], rows1, gsem)
                pltpu.async_copy(rows0, acc.at[id0], ssem, add=True)
                wait_gather(rows1)
                wait_scatter(rows0)
                pltpu.async_copy(rows1, acc.at[id1], ssem, add=True)
                return 0

            lax.fori_loop(0, cpw // 2, pair, 0)
            wait_scatter(rows1)
            plsc.subcore_barrier()

            def wb(i, _):
                rs = pl.ds(r0 + i * CE, CE)
                pltpu.sync_copy(acc.at[rs, :], rows0)
                pltpu.sync_copy(rows0, out_hbm.at[cid, kk, rs, :])
                return 0

            lax.fori_loop(0, rpt // CE, wb, 0)

    return scatter_kernel


def _dinv_of(deg_ref):
    deg = deg_ref[0, :] + deg_ref[1, :] + 1.0
    return lax.rsqrt(deg)[:, None]


def _split(h, h_ref):
    dh = h.shape[1] // 2
    h_ref[0] = h[:, :dh]
    h_ref[1] = h[:, dh:]


def _merge(p_ref, hprev_ref):
    return jnp.concatenate(
        [p_ref[0, 0] + p_ref[1, 0] + hprev_ref[0],
         p_ref[0, 1] + p_ref[1, 1] + hprev_ref[1]],
        axis=-1,
    )


def _tc_first(deg_ref, x_ref, w_ref, h_ref):
    dinv = _dinv_of(deg_ref)
    h = jnp.dot(x_ref[...], w_ref[...], preferred_element_type=jnp.float32)
    _split(h * dinv, h_ref)


def _tc_mid(n_real, eps, p_ref, hprev_ref, deg_ref, b_ref, g_ref, be_ref, w_ref, h_ref):
    n_pad = hprev_ref.shape[1]
    dinv = _dinv_of(deg_ref)
    agg = _merge(p_ref, hprev_ref)
    y = agg * dinv + b_ref[...][None, :]
    rid = lax.broadcasted_iota(jnp.int32, (n_pad, 1), 0)
    mask = rid < n_real
    y = jnp.where(mask, y, 0.0)
    mu = jnp.sum(y, axis=0, keepdims=True) / n_real
    cent = jnp.where(mask, y - mu, 0.0)
    var = jnp.sum(cent * cent, axis=0, keepdims=True) / n_real
    yn = g_ref[...][None, :] * cent * lax.rsqrt(var + eps) + be_ref[...][None, :]
    z = jnp.where(mask, jnp.maximum(yn, 0.0), 0.0)
    h = jnp.dot(z, w_ref[...], preferred_element_type=jnp.float32) * dinv
    _split(h, h_ref)


def _tc_last(n_real, p_ref, hprev_ref, deg_ref, b_ref, out_ref):
    dinv = _dinv_of(deg_ref)
    agg = _merge(p_ref, hprev_ref)
    y = agg * dinv + b_ref[...][None, :]
    out_ref[...] = y[:n_real, :]


def kernel(x, edge_index, W1, b1, g1, be1, W2, b2, g2, be2, W3, b3):
    n, d = x.shape
    e = edge_index.shape[1]
    n_acc = ((n + 1 + (NS * L) - 1) // (NS * L)) * (NS * L)
    gran = NW * CE * 2
    e_pad = ((e + gran - 1) // gran) * gran

    pad = jnp.full((e_pad - e,), n, dtype=jnp.int32)
    src = jnp.concatenate([edge_index[0], pad])
    dst = jnp.concatenate([edge_index[1], pad])
    x_pad = jnp.pad(x, ((0, n_acc - n), (0, 0)))

    deg_kernel = _make_deg_kernel(e_pad, n_acc)
    scatter_kernel = _make_scatter_kernel(e_pad, n_acc, d)

    deg2 = deg_kernel(dst)

    hshape = jax.ShapeDtypeStruct((2, n_acc, d // 2), jnp.float32)
    h1 = pl.pallas_call(
        _tc_first,
        out_shape=hshape,
    )(deg2, x_pad, W1)

    p1 = scatter_kernel(h1, src, dst)
    h2 = pl.pallas_call(
        functools.partial(_tc_mid, n, 1e-5),
        out_shape=hshape,
    )(p1, h1, deg2, b1, g1, be1, W2)

    p2 = scatter_kernel(h2, src, dst)
    h3 = pl.pallas_call(
        functools.partial(_tc_mid, n, 1e-5),
        out_shape=hshape,
    )(p2, h2, deg2, b2, g2, be2, W3)

    p3 = scatter_kernel(h3, src, dst)
    out = pl.pallas_call(
        functools.partial(_tc_last, n),
        out_shape=jax.ShapeDtypeStruct((n, d), jnp.float32),
    )(p3, h3, deg2, b3)
    return out
